# Initial kernel scaffold; baseline (speedup 1.0000x reference)
import jax
import jax.numpy as jnp
from jax.experimental import pallas as pl


def kernel(TM, link_capacity, link_indices, path_indices, sequ_indices, n_paths, n_links, n_total, paths, params):
    raise NotImplementedError("write your pallas kernel here")


# speedup vs baseline: 2.0719x; 2.0719x over previous
import functools

import jax
import jax.numpy as jnp
from jax import lax
from jax.experimental import pallas as pl
from jax.experimental.pallas import tpu as pltpu
from jax.experimental.pallas import tpu_sc as plsc

N_PATHS = 20000
N_LINKS = 10000
N_EDGES = 320000
F = 32
HID = 256
N_ITERS = 4

NC = 2
NS = 16
NW = NC * NS
CH = 128
NCHE = 80
EPW = NCHE * CH
E_PAD = NW * EPW

NP_PAD = 20480
NL_PAD = 10240

_f32 = jnp.float32


def _sc_gather(link_states, path_states, lidx3, pidx3):
    mesh = plsc.VectorSubcoreMesh(core_axis_name="c", subcore_axis_name="s", num_cores=NC, num_subcores=NS)

    @functools.partial(
        pl.kernel,
        out_type=(
            jax.ShapeDtypeStruct((E_PAD, F), _f32),
            jax.ShapeDtypeStruct((E_PAD, F), _f32),
        ),
        mesh=mesh,
        scratch_types=[
            pltpu.VMEM((NCHE, CH), jnp.int32),
            pltpu.VMEM((NCHE, CH), jnp.int32),
            pltpu.VMEM((CH, F), _f32),
            pltpu.VMEM((CH, F), _f32),
            pltpu.SemaphoreType.DMA,
        ],
        compiler_params=pltpu.CompilerParams(use_tc_tiling_on_sc=False),
    )
    def gather_kernel(lt_hbm, pt_hbm, lidx_hbm, pidx_hbm, ls_out, ps_out,
                      lidx_v, pidx_v, lrow, prow, gsem):
        c = lax.axis_index("c")
        s = lax.axis_index("s")
        wid = s * NC + c
        pltpu.sync_copy(lidx_hbm.at[wid], lidx_v)
        pltpu.sync_copy(pidx_hbm.at[wid], pidx_v)
        base = wid * EPW

        def body(j, carry):
            cl = pltpu.async_copy(lt_hbm.at[lidx_v.at[j]], lrow, gsem)
            cp = pltpu.async_copy(pt_hbm.at[pidx_v.at[j]], prow, gsem)
            cl.wait()
            cp.wait()
            pltpu.sync_copy(lrow, ls_out.at[pl.ds(base + j * CH, CH)])
            pltpu.sync_copy(prow, ps_out.at[pl.ds(base + j * CH, CH)])
            return carry

        lax.fori_loop(0, NCHE, body, 0)

    return gather_kernel(link_states, path_states, lidx3, pidx3)


def _sc_scatter(msg_p, msg_l, pidx3, lidx3):
    mesh = plsc.VectorSubcoreMesh(core_axis_name="c", subcore_axis_name="s", num_cores=NC, num_subcores=NS)
    ZP = NP_PAD // CH // NS
    ZL = NL_PAD // CH // NS

    @functools.partial(
        pl.kernel,
        out_type=(
            jax.ShapeDtypeStruct((NC, NP_PAD, F), _f32),
            jax.ShapeDtypeStruct((NC, NL_PAD, F), _f32),
        ),
        mesh=mesh,
        scratch_types=[
            pltpu.VMEM((NCHE, CH), jnp.int32),
            pltpu.VMEM((NCHE, CH), jnp.int32),
            pltpu.VMEM((CH, F), _f32),
            pltpu.VMEM((CH, F), _f32),
            pltpu.VMEM_SHARED((NP_PAD, F), _f32),
            pltpu.VMEM_SHARED((NL_PAD, F), _f32),
        ],
        compiler_params=pltpu.CompilerParams(use_tc_tiling_on_sc=False),
    )
    def scatter_kernel(mp_hbm, ml_hbm, pidx_hbm, lidx_hbm, outp_hbm, outl_hbm,
                       pidx_v, lidx_v, mbuf, zbuf, accp, accl):
        c = lax.axis_index("c")
        s = lax.axis_index("s")
        wid = s * NC + c

        def zfill(i, carry):
            zbuf[i, pl.ds(0, 16)] = jnp.zeros((16,), _f32)
            zbuf[i, pl.ds(16, 16)] = jnp.zeros((16,), _f32)
            return carry

        lax.fori_loop(0, CH, zfill, 0)

        def zp(i, carry):
            pltpu.sync_copy(zbuf, accp.at[pl.ds((s * ZP + i) * CH, CH)])
            return carry

        lax.fori_loop(0, ZP, zp, 0)

        def zl(i, carry):
            pltpu.sync_copy(zbuf, accl.at[pl.ds((s * ZL + i) * CH, CH)])
            return carry

        lax.fori_loop(0, ZL, zl, 0)
        plsc.subcore_barrier()

        pltpu.sync_copy(pidx_hbm.at[wid], pidx_v)
        pltpu.sync_copy(lidx_hbm.at[wid], lidx_v)
        base = wid * EPW

        def body(j, carry):
            pltpu.sync_copy(mp_hbm.at[pl.ds(base + j * CH, CH)], mbuf)
            pltpu.sync_copy(mbuf, accp.at[pidx_v.at[j]], add=True)
            pltpu.sync_copy(ml_hbm.at[pl.ds(base + j * CH, CH)], mbuf)
            pltpu.sync_copy(mbuf, accl.at[lidx_v.at[j]], add=True)
            return carry

        lax.fori_loop(0, NCHE, body, 0)
        plsc.subcore_barrier()

        def outp(i, carry):
            r = (s * ZP + i) * CH
            pltpu.sync_copy(accp.at[pl.ds(r, CH)], mbuf)
            pltpu.sync_copy(mbuf, outp_hbm.at[c, pl.ds(r, CH)])
            return carry

        lax.fori_loop(0, ZP, outp, 0)

        def outl(i, carry):
            r = (s * ZL + i) * CH
            pltpu.sync_copy(accl.at[pl.ds(r, CH)], mbuf)
            pltpu.sync_copy(mbuf, outl_hbm.at[c, pl.ds(r, CH)])
            return carry

        lax.fori_loop(0, ZL, outl, 0)

    return scatter_kernel(msg_p, msg_l, pidx3, lidx3)


def _tc_msg_mlp(ls_e, ps_e, w):
    BE = 2048
    grid = (E_PAD // BE,)

    def mlp_body(ls_ref, ps_ref, al_ref, ap_ref, b1_ref, w2p_ref, b2p_ref,
                 w2l_ref, b2l_ref, w3p_ref, b3p_ref, w3l_ref, b3l_ref,
                 mp_ref, ml_ref):
        h1 = jnp.maximum(
            jnp.dot(ls_ref[...], al_ref[...], preferred_element_type=_f32)
            + jnp.dot(ps_ref[...], ap_ref[...], preferred_element_type=_f32)
            + b1_ref[...], 0.0)
        h2p = jnp.maximum(
            jnp.dot(h1[:, :HID], w2p_ref[...], preferred_element_type=_f32)
            + b2p_ref[...], 0.0)
        h2l = jnp.maximum(
            jnp.dot(h1[:, HID:], w2l_ref[...], preferred_element_type=_f32)
            + b2l_ref[...], 0.0)
        mp_ref[...] = (jnp.dot(h2p, w3p_ref[...], preferred_element_type=_f32)
                       + b3p_ref[...])
        ml_ref[...] = (jnp.dot(h2l, w3l_ref[...], preferred_element_type=_f32)
                       + b3l_ref[...])

    edge_spec = pl.BlockSpec((BE, F), lambda i: (i, 0))
    wspec = lambda shp: pl.BlockSpec(shp, lambda i: (0, 0))
    return pl.pallas_call(
        mlp_body,
        grid=grid,
        in_specs=[
            edge_spec, edge_spec,
            wspec((F, 2 * HID)), wspec((F, 2 * HID)), wspec((1, 2 * HID)),
            wspec((HID, HID)), wspec((1, HID)),
            wspec((HID, HID)), wspec((1, HID)),
            wspec((HID, F)), wspec((1, F)),
            wspec((HID, F)), wspec((1, F)),
        ],
        out_specs=[edge_spec, edge_spec],
        out_shape=[
            jax.ShapeDtypeStruct((E_PAD, F), _f32),
            jax.ShapeDtypeStruct((E_PAD, F), _f32),
        ],
        compiler_params=pltpu.CompilerParams(
            dimension_semantics=("arbitrary",)),
    )(ls_e, ps_e, w["Al"], w["Ap"], w["b1"], w["W2p"], w["b2p"],
      w["W2l"], w["b2l"], w["W3p"], w["b3p"], w["W3l"], w["b3l"])


def _tc_gru(aggp, hp, aggl, hl, w):
    BP, BL = 1024, 512
    grid = (NP_PAD // BP,)

    def gru_body(aggp_ref, hp_ref, aggl_ref, hl_ref,
                 pwir, pwiz, pwin, pwhr, pwhz, pwhn, pbr, pbz, pbin, pbhn,
                 lwir, lwiz, lwin, lwhr, lwhz, lwhn, lbr, lbz, lbin, lbhn,
                 hp_out, hl_out):
        def gru(x, h, wir, wiz, win, whr, whz, whn, br, bz, bin_, bhn):
            r = jax.nn.sigmoid(
                jnp.dot(x, wir[...], preferred_element_type=_f32)
                + jnp.dot(h, whr[...], preferred_element_type=_f32) + br[...])
            z = jax.nn.sigmoid(
                jnp.dot(x, wiz[...], preferred_element_type=_f32)
                + jnp.dot(h, whz[...], preferred_element_type=_f32) + bz[...])
            n = jnp.tanh(
                jnp.dot(x, win[...], preferred_element_type=_f32) + bin_[...]
                + r * (jnp.dot(h, whn[...], preferred_element_type=_f32)
                       + bhn[...]))
            return (1.0 - z) * n + z * h

        xp = aggp_ref[0] + aggp_ref[1]
        hp = hp_ref[...]
        hp_out[...] = gru(xp, hp, pwir, pwiz, pwin, pwhr, pwhz, pwhn,
                          pbr, pbz, pbin, pbhn)
        xl = aggl_ref[0] + aggl_ref[1]
        hl = hl_ref[...]
        hl_out[...] = gru(xl, hl, lwir, lwiz, lwin, lwhr, lwhz, lwhn,
                          lbr, lbz, lbin, lbhn)

    wspec = lambda: pl.BlockSpec((F, F), lambda i: (0, 0))
    bspec = lambda: pl.BlockSpec((1, F), lambda i: (0, 0))
    wnames = ["Wir", "Wiz", "Win", "Whr", "Whz", "Whn", "br", "bz", "bin", "bhn"]
    return pl.pallas_call(
        gru_body,
        grid=grid,
        in_specs=[
            pl.BlockSpec((NC, BP, F), lambda i: (0, i, 0)),
            pl.BlockSpec((BP, F), lambda i: (i, 0)),
            pl.BlockSpec((NC, BL, F), lambda i: (0, i, 0)),
            pl.BlockSpec((BL, F), lambda i: (i, 0)),
        ] + [wspec() if n[0] == "W" else bspec() for n in wnames] * 2,
        out_specs=[
            pl.BlockSpec((BP, F), lambda i: (i, 0)),
            pl.BlockSpec((BL, F), lambda i: (i, 0)),
        ],
        out_shape=[
            jax.ShapeDtypeStruct((NP_PAD, F), _f32),
            jax.ShapeDtypeStruct((NL_PAD, F), _f32),
        ],
        compiler_params=pltpu.CompilerParams(
            dimension_semantics=("arbitrary",)),
    )(aggp, hp, aggl, hl,
      *[w["p" + n] for n in wnames], *[w["l" + n] for n in wnames])


def _tc_readout(hp, w):
    BR = 2000
    grid = (N_PATHS // BR,)

    def ro_body(hp_ref, w1, b1, w2, b2, w3, b3, out_ref):
        h = jnp.maximum(
            jnp.dot(hp_ref[...], w1[...], preferred_element_type=_f32)
            + b1[...], 0.0)
        h = jnp.maximum(
            jnp.dot(h, w2[...], preferred_element_type=_f32) + b2[...], 0.0)
        out_ref[...] = jnp.dot(h, w3[...], preferred_element_type=_f32) + b3[...]

    wspec = lambda shp: pl.BlockSpec(shp, lambda i: (0, 0))
    return pl.pallas_call(
        ro_body,
        grid=grid,
        in_specs=[
            pl.BlockSpec((BR, F), lambda i: (i, 0)),
            wspec((F, HID)), wspec((1, HID)),
            wspec((HID, HID)), wspec((1, HID)),
            wspec((HID, 1)), wspec((1, 1)),
        ],
        out_specs=pl.BlockSpec((BR, 1), lambda i: (i, 0)),
        out_shape=jax.ShapeDtypeStruct((N_PATHS, 1), _f32),
        compiler_params=pltpu.CompilerParams(
            dimension_semantics=("arbitrary",)),
    )(hp, w["roW1"], w["rob1"], w["roW2"], w["rob2"], w["roW3"], w["rob3"])


def kernel(TM, link_capacity, link_indices, path_indices, sequ_indices,
           n_paths, n_links, n_total, paths, params):
    p = params
    w = {
        "Al": jnp.concatenate([p["pm_W1"][:, :F].T, p["lm_W1"][:, F:].T], axis=1),
        "Ap": jnp.concatenate([p["pm_W1"][:, F:].T, p["lm_W1"][:, :F].T], axis=1),
        "b1": jnp.concatenate([p["pm_b1"], p["lm_b1"]]).reshape(1, 2 * HID),
        "W2p": p["pm_W2"].T, "b2p": p["pm_b2"].reshape(1, HID),
        "W2l": p["lm_W2"].T, "b2l": p["lm_b2"].reshape(1, HID),
        "W3p": p["pm_W3"].T, "b3p": p["pm_b3"].reshape(1, F),
        "W3l": p["lm_W3"].T, "b3l": p["lm_b3"].reshape(1, F),
        "roW1": p["ro_W1"].T, "rob1": p["ro_b1"].reshape(1, HID),
        "roW2": p["ro_W2"].T, "rob2": p["ro_b2"].reshape(1, HID),
        "roW3": p["ro_W3"].T, "rob3": p["ro_b3"].reshape(1, 1),
    }
    for pre, tag in (("pg", "p"), ("lg", "l")):
        Wih, Whh = p[pre + "_Wih"], p[pre + "_Whh"]
        bih, bhh = p[pre + "_bih"], p[pre + "_bhh"]
        w[tag + "Wir"] = Wih[:F].T
        w[tag + "Wiz"] = Wih[F:2 * F].T
        w[tag + "Win"] = Wih[2 * F:].T
        w[tag + "Whr"] = Whh[:F].T
        w[tag + "Whz"] = Whh[F:2 * F].T
        w[tag + "Whn"] = Whh[2 * F:].T
        w[tag + "br"] = (bih[:F] + bhh[:F]).reshape(1, F)
        w[tag + "bz"] = (bih[F:2 * F] + bhh[F:2 * F]).reshape(1, F)
        w[tag + "bin"] = bih[2 * F:].reshape(1, F)
        w[tag + "bhn"] = bhh[2 * F:].reshape(1, F)

    pad = E_PAD - N_EDGES
    lidx3 = jnp.concatenate(
        [link_indices, jnp.full((pad,), N_LINKS, jnp.int32)]).reshape(NW, NCHE, CH)
    pidx3 = jnp.concatenate(
        [path_indices, jnp.full((pad,), N_PATHS, jnp.int32)]).reshape(NW, NCHE, CH)

    link_states = jnp.zeros((NL_PAD, F), _f32).at[:N_LINKS, 0].set(link_capacity)
    path_states = jnp.zeros((NP_PAD, F), _f32).at[:N_PATHS, 0].set(TM)

    for _ in range(N_ITERS):
        ls_e, ps_e = _sc_gather(link_states, path_states, lidx3, pidx3)
        msg_p, msg_l = _tc_msg_mlp(ls_e, ps_e, w)
        aggp, aggl = _sc_scatter(msg_p, msg_l, pidx3, lidx3)
        path_states, link_states = _tc_gru(aggp, path_states, aggl, link_states, w)

    return _tc_readout(path_states, w)


# speedup vs baseline: 3.8110x; 1.8394x over previous
import functools

import jax
import jax.numpy as jnp
from jax import lax
from jax.experimental import pallas as pl
from jax.experimental.pallas import tpu as pltpu
from jax.experimental.pallas import tpu_sc as plsc

N_PATHS = 20000
N_LINKS = 10000
N_EDGES = 320000
F = 32
HID = 256
N_ITERS = 4

NC = 2
NS = 16
NW = NC * NS
CH = 128
NCHE = 80
EPW = NCHE * CH
E_PAD = NW * EPW

EQ = E_PAD // 4
GW = NW // 4

NP_PAD = 20480
NL_PAD = 10240

_f32 = jnp.float32


def _sc_gather(link_states, path_states, lidx3, pidx3):
    mesh = plsc.VectorSubcoreMesh(core_axis_name="c", subcore_axis_name="s", num_cores=NC, num_subcores=NS)

    @functools.partial(
        pl.kernel,
        out_type=(
            jax.ShapeDtypeStruct((EQ, 128), _f32),
            jax.ShapeDtypeStruct((EQ, 128), _f32),
        ),
        mesh=mesh,
        scratch_types=[
            pltpu.VMEM((NCHE, CH), jnp.int32),
            pltpu.VMEM((NCHE, CH), jnp.int32),
            pltpu.VMEM((CH, F), _f32),
            pltpu.VMEM((CH, F), _f32),
            pltpu.VMEM((CH, F), _f32),
            pltpu.VMEM((CH, F), _f32),
            pltpu.SemaphoreType.DMA,
            pltpu.SemaphoreType.DMA,
            pltpu.SemaphoreType.DMA,
            pltpu.SemaphoreType.DMA,
        ],
        compiler_params=pltpu.CompilerParams(use_tc_tiling_on_sc=False),
    )
    def gather_kernel(lt_hbm, pt_hbm, lidx_hbm, pidx_hbm, ls_out, ps_out,
                      lidx_v, pidx_v, lrow0, prow0, lrow1, prow1,
                      gsem0, gsem1, wsem0, wsem1):
        c = lax.axis_index("c")
        s = lax.axis_index("s")
        wid = s * NC + c
        pltpu.sync_copy(lidx_hbm.at[wid], lidx_v)
        pltpu.sync_copy(pidx_hbm.at[wid], pidx_v)
        r0 = lax.rem(wid, GW) * EPW
        g0 = lax.div(wid, GW) * F
        NT = NCHE // 2

        def fire_g(j, lrow, prow, sem):
            pltpu.async_copy(lt_hbm.at[lidx_v.at[j]], lrow, sem)
            pltpu.async_copy(pt_hbm.at[pidx_v.at[j]], prow, sem)

        def wait_g(j, lrow, prow, sem):
            pltpu.make_async_copy(lt_hbm.at[lidx_v.at[j]], lrow, sem).wait()
            pltpu.make_async_copy(pt_hbm.at[pidx_v.at[j]], prow, sem).wait()

        def fire_w(j, lrow, prow, sem):
            dst = (pl.ds(r0 + j * CH, CH), pl.ds(g0, F))
            pltpu.async_copy(lrow, ls_out.at[dst], sem)
            pltpu.async_copy(prow, ps_out.at[dst], sem)

        def wait_w(j, lrow, prow, sem):
            dst = (pl.ds(r0 + j * CH, CH), pl.ds(g0, F))
            pltpu.make_async_copy(lrow, ls_out.at[dst], sem).wait()
            pltpu.make_async_copy(prow, ps_out.at[dst], sem).wait()

        fire_g(0, lrow0, prow0, gsem0)

        def body(t, carry):
            j0 = 2 * t
            j1 = j0 + 1
            wait_g(j0, lrow0, prow0, gsem0)

            @pl.when(t > 0)
            def _():
                wait_w(j1 - 2, lrow1, prow1, wsem1)

            fire_g(j1, lrow1, prow1, gsem1)
            fire_w(j0, lrow0, prow0, wsem0)
            wait_g(j1, lrow1, prow1, gsem1)
            wait_w(j0, lrow0, prow0, wsem0)

            @pl.when(t < NT - 1)
            def _():
                fire_g(j0 + 2, lrow0, prow0, gsem0)

            fire_w(j1, lrow1, prow1, wsem1)
            return carry

        lax.fori_loop(0, NT, body, 0)
        wait_w(NCHE - 1, lrow1, prow1, wsem1)

    return gather_kernel(link_states, path_states, lidx3, pidx3)


def _sc_scatter(msg_p, msg_l, pidx3, lidx3):
    mesh = plsc.VectorSubcoreMesh(core_axis_name="c", subcore_axis_name="s", num_cores=NC, num_subcores=NS)
    ZP = NP_PAD // CH // NS
    ZL = NL_PAD // CH // NS

    @functools.partial(
        pl.kernel,
        out_type=(
            jax.ShapeDtypeStruct((NC, NP_PAD, F), _f32),
            jax.ShapeDtypeStruct((NC, NL_PAD, F), _f32),
        ),
        mesh=mesh,
        scratch_types=[
            pltpu.VMEM((NCHE, CH), jnp.int32),
            pltpu.VMEM((NCHE, CH), jnp.int32),
            pltpu.VMEM((CH, F), _f32),
            pltpu.VMEM((CH, F), _f32),
            pltpu.VMEM((CH, F), _f32),
            pltpu.VMEM((CH, F), _f32),
            pltpu.VMEM((CH, F), _f32),
            pltpu.VMEM_SHARED((NP_PAD, F), _f32),
            pltpu.VMEM_SHARED((NL_PAD, F), _f32),
            pltpu.SemaphoreType.DMA,
            pltpu.SemaphoreType.DMA,
        ],
        compiler_params=pltpu.CompilerParams(use_tc_tiling_on_sc=False),
    )
    def scatter_kernel(mp_hbm, ml_hbm, pidx_hbm, lidx_hbm, outp_hbm, outl_hbm,
                       pidx_v, lidx_v, mp0, ml0, mp1, ml1, zbuf, accp, accl,
                       msem0, msem1):
        c = lax.axis_index("c")
        s = lax.axis_index("s")
        wid = s * NC + c

        def zfill(i, carry):
            zbuf[i, pl.ds(0, 16)] = jnp.zeros((16,), _f32)
            zbuf[i, pl.ds(16, 16)] = jnp.zeros((16,), _f32)
            return carry

        lax.fori_loop(0, CH, zfill, 0)

        def zp(i, carry):
            pltpu.sync_copy(zbuf, accp.at[pl.ds((s * ZP + i) * CH, CH)])
            return carry

        lax.fori_loop(0, ZP, zp, 0)

        def zl(i, carry):
            pltpu.sync_copy(zbuf, accl.at[pl.ds((s * ZL + i) * CH, CH)])
            return carry

        lax.fori_loop(0, ZL, zl, 0)

        pltpu.sync_copy(pidx_hbm.at[wid], pidx_v)
        pltpu.sync_copy(lidx_hbm.at[wid], lidx_v)
        r0 = lax.rem(wid, GW) * EPW
        g0 = lax.div(wid, GW) * F
        plsc.subcore_barrier()

        NT = NCHE // 2

        def fire_m(j, bp, bl, sem):
            srcs = (pl.ds(r0 + j * CH, CH), pl.ds(g0, F))
            pltpu.async_copy(mp_hbm.at[srcs], bp, sem)
            pltpu.async_copy(ml_hbm.at[srcs], bl, sem)

        def wait_m(j, bp, bl, sem):
            srcs = (pl.ds(r0 + j * CH, CH), pl.ds(g0, F))
            pltpu.make_async_copy(mp_hbm.at[srcs], bp, sem).wait()
            pltpu.make_async_copy(ml_hbm.at[srcs], bl, sem).wait()

        fire_m(0, mp0, ml0, msem0)

        def body(t, carry):
            j0 = 2 * t
            j1 = j0 + 1
            wait_m(j0, mp0, ml0, msem0)
            fire_m(j1, mp1, ml1, msem1)
            pltpu.sync_copy(mp0, accp.at[pidx_v.at[j0]], add=True)
            pltpu.sync_copy(ml0, accl.at[lidx_v.at[j0]], add=True)
            wait_m(j1, mp1, ml1, msem1)

            @pl.when(t < NT - 1)
            def _():
                fire_m(j0 + 2, mp0, ml0, msem0)

            pltpu.sync_copy(mp1, accp.at[pidx_v.at[j1]], add=True)
            pltpu.sync_copy(ml1, accl.at[lidx_v.at[j1]], add=True)
            return carry

        lax.fori_loop(0, NT, body, 0)
        plsc.subcore_barrier()

        def outp(i, carry):
            r = (s * ZP + i) * CH
            pltpu.sync_copy(accp.at[pl.ds(r, CH)], mp0)
            pltpu.sync_copy(mp0, outp_hbm.at[c, pl.ds(r, CH)])
            return carry

        lax.fori_loop(0, ZP, outp, 0)

        def outl(i, carry):
            r = (s * ZL + i) * CH
            pltpu.sync_copy(accl.at[pl.ds(r, CH)], mp0)
            pltpu.sync_copy(mp0, outl_hbm.at[c, pl.ds(r, CH)])
            return carry

        lax.fori_loop(0, ZL, outl, 0)

    return scatter_kernel(msg_p, msg_l, pidx3, lidx3)


def _tc_msg_mlp(ls_e, ps_e, w):
    BQ = 512
    grid = (EQ // BQ,)

    def mlp_body(ls_ref, ps_ref, w1p_ref, w1l_ref, b1p_ref, b1l_ref,
                 w2p_ref, b2p_ref, w2l_ref, b2l_ref, w3p_ref, b3p_ref,
                 w3l_ref, b3l_ref, mp_ref, ml_ref):
        for k in range(4):
            sl = slice(k * F, (k + 1) * F)
            ls = ls_ref[:, sl]
            ps = ps_ref[:, sl]
            xp = jnp.concatenate([ls, ps], axis=1)
            xl = jnp.concatenate([ps, ls], axis=1)
            h1p = jnp.maximum(
                jnp.dot(xp, w1p_ref[...], preferred_element_type=_f32)
                + b1p_ref[...], 0.0)
            h1l = jnp.maximum(
                jnp.dot(xl, w1l_ref[...], preferred_element_type=_f32)
                + b1l_ref[...], 0.0)
            h2p = jnp.maximum(
                jnp.dot(h1p, w2p_ref[...], preferred_element_type=_f32)
                + b2p_ref[...], 0.0)
            h2l = jnp.maximum(
                jnp.dot(h1l, w2l_ref[...], preferred_element_type=_f32)
                + b2l_ref[...], 0.0)
            mp_ref[:, sl] = (jnp.dot(h2p, w3p_ref[...],
                                     preferred_element_type=_f32)
                             + b3p_ref[...])
            ml_ref[:, sl] = (jnp.dot(h2l, w3l_ref[...],
                                     preferred_element_type=_f32)
                             + b3l_ref[...])

    edge_spec = pl.BlockSpec((BQ, 128), lambda i: (i, 0))
    wspec = lambda shp: pl.BlockSpec(shp, lambda i: (0, 0))
    return pl.pallas_call(
        mlp_body,
        grid=grid,
        in_specs=[
            edge_spec, edge_spec,
            wspec((2 * F, HID)), wspec((2 * F, HID)),
            wspec((1, HID)), wspec((1, HID)),
            wspec((HID, HID)), wspec((1, HID)),
            wspec((HID, HID)), wspec((1, HID)),
            wspec((HID, F)), wspec((1, F)),
            wspec((HID, F)), wspec((1, F)),
        ],
        out_specs=[edge_spec, edge_spec],
        out_shape=[
            jax.ShapeDtypeStruct((EQ, 128), _f32),
            jax.ShapeDtypeStruct((EQ, 128), _f32),
        ],
        compiler_params=pltpu.CompilerParams(
            dimension_semantics=("arbitrary",)),
    )(ls_e, ps_e, w["W1p"], w["W1l"], w["b1p"], w["b1l"], w["W2p"],
      w["b2p"], w["W2l"], w["b2l"], w["W3p"], w["b3p"], w["W3l"], w["b3l"])


def _tc_gru(aggp, hp, aggl, hl, w):
    BP, BL = 1024, 512
    grid = (NP_PAD // BP,)

    def gru_body(aggp_ref, hp_ref, aggl_ref, hl_ref,
                 pwih, pwhh, pbih, pbhh, lwih, lwhh, lbih, lbhh,
                 hp_out, hl_out):
        def gru(x, h, wih, whh, bih, bhh):
            gi = jnp.dot(x, wih[...], preferred_element_type=_f32) + bih[...]
            gh = jnp.dot(h, whh[...], preferred_element_type=_f32) + bhh[...]
            r = jax.nn.sigmoid(gi[:, :F] + gh[:, :F])
            z = jax.nn.sigmoid(gi[:, F:2 * F] + gh[:, F:2 * F])
            n = jnp.tanh(gi[:, 2 * F:] + r * gh[:, 2 * F:])
            return (1.0 - z) * n + z * h

        xp = aggp_ref[0] + aggp_ref[1]
        hp_out[...] = gru(xp, hp_ref[...], pwih, pwhh, pbih, pbhh)
        xl = aggl_ref[0] + aggl_ref[1]
        hl_out[...] = gru(xl, hl_ref[...], lwih, lwhh, lbih, lbhh)

    wspec = lambda: pl.BlockSpec((F, 3 * F), lambda i: (0, 0))
    bspec = lambda: pl.BlockSpec((1, 3 * F), lambda i: (0, 0))
    return pl.pallas_call(
        gru_body,
        grid=grid,
        in_specs=[
            pl.BlockSpec((NC, BP, F), lambda i: (0, i, 0)),
            pl.BlockSpec((BP, F), lambda i: (i, 0)),
            pl.BlockSpec((NC, BL, F), lambda i: (0, i, 0)),
            pl.BlockSpec((BL, F), lambda i: (i, 0)),
            wspec(), wspec(), bspec(), bspec(),
            wspec(), wspec(), bspec(), bspec(),
        ],
        out_specs=[
            pl.BlockSpec((BP, F), lambda i: (i, 0)),
            pl.BlockSpec((BL, F), lambda i: (i, 0)),
        ],
        out_shape=[
            jax.ShapeDtypeStruct((NP_PAD, F), _f32),
            jax.ShapeDtypeStruct((NL_PAD, F), _f32),
        ],
        compiler_params=pltpu.CompilerParams(
            dimension_semantics=("arbitrary",)),
    )(aggp, hp, aggl, hl,
      w["pWih"], w["pWhh"], w["pbih"], w["pbhh"],
      w["lWih"], w["lWhh"], w["lbih"], w["lbhh"])


def _tc_readout(hp, w):
    BR = 2000
    grid = (N_PATHS // BR,)

    def ro_body(hp_ref, w1, b1, w2, b2, w3, b3, out_ref):
        h = jnp.maximum(
            jnp.dot(hp_ref[...], w1[...], preferred_element_type=_f32)
            + b1[...], 0.0)
        h = jnp.maximum(
            jnp.dot(h, w2[...], preferred_element_type=_f32) + b2[...], 0.0)
        out_ref[...] = jnp.dot(h, w3[...], preferred_element_type=_f32) + b3[...]

    wspec = lambda shp: pl.BlockSpec(shp, lambda i: (0, 0))
    return pl.pallas_call(
        ro_body,
        grid=grid,
        in_specs=[
            pl.BlockSpec((BR, F), lambda i: (i, 0)),
            wspec((F, HID)), wspec((1, HID)),
            wspec((HID, HID)), wspec((1, HID)),
            wspec((HID, 1)), wspec((1, 1)),
        ],
        out_specs=pl.BlockSpec((BR, 1), lambda i: (i, 0)),
        out_shape=jax.ShapeDtypeStruct((N_PATHS, 1), _f32),
        compiler_params=pltpu.CompilerParams(
            dimension_semantics=("arbitrary",)),
    )(hp, w["roW1"], w["rob1"], w["roW2"], w["rob2"], w["roW3"], w["rob3"])


def kernel(TM, link_capacity, link_indices, path_indices, sequ_indices,
           n_paths, n_links, n_total, paths, params):
    p = params
    w = {
        "W1p": p["pm_W1"].T,
        "W1l": p["lm_W1"].T,
        "b1p": p["pm_b1"].reshape(1, HID),
        "b1l": p["lm_b1"].reshape(1, HID),
        "W2p": p["pm_W2"].T, "b2p": p["pm_b2"].reshape(1, HID),
        "W2l": p["lm_W2"].T, "b2l": p["lm_b2"].reshape(1, HID),
        "W3p": p["pm_W3"].T, "b3p": p["pm_b3"].reshape(1, F),
        "W3l": p["lm_W3"].T, "b3l": p["lm_b3"].reshape(1, F),
        "roW1": p["ro_W1"].T, "rob1": p["ro_b1"].reshape(1, HID),
        "roW2": p["ro_W2"].T, "rob2": p["ro_b2"].reshape(1, HID),
        "roW3": p["ro_W3"].T, "rob3": p["ro_b3"].reshape(1, 1),
    }
    for pre, tag in (("pg", "p"), ("lg", "l")):
        w[tag + "Wih"] = p[pre + "_Wih"].T
        w[tag + "Whh"] = p[pre + "_Whh"].T
        w[tag + "bih"] = p[pre + "_bih"].reshape(1, 3 * F)
        w[tag + "bhh"] = p[pre + "_bhh"].reshape(1, 3 * F)

    pad = E_PAD - N_EDGES
    lidx3 = jnp.concatenate(
        [link_indices, jnp.full((pad,), N_LINKS, jnp.int32)]).reshape(NW, NCHE, CH)
    pidx3 = jnp.concatenate(
        [path_indices, jnp.full((pad,), N_PATHS, jnp.int32)]).reshape(NW, NCHE, CH)

    link_states = jnp.zeros((NL_PAD, F), _f32).at[:N_LINKS, 0].set(link_capacity)
    path_states = jnp.zeros((NP_PAD, F), _f32).at[:N_PATHS, 0].set(TM)

    for _ in range(N_ITERS):
        ls_e, ps_e = _sc_gather(link_states, path_states, lidx3, pidx3)
        msg_p, msg_l = _tc_msg_mlp(ls_e, ps_e, w)
        aggp, aggl = _sc_scatter(msg_p, msg_l, pidx3, lidx3)
        path_states, link_states = _tc_gru(aggp, path_states, aggl, link_states, w)

    return _tc_readout(path_states, w)
